# baseline (speedup 1.0000x reference)
import functools

import jax
import jax.numpy as jnp
from jax import lax
from jax.experimental import pallas as pl
from jax.experimental.pallas import tpu as pltpu
from jax.experimental.pallas import tpu_sc as plsc

NW = 32
LANES = 16


def _make_word_kernel(bsz, seq, d_word):
    assert bsz == 128 * NW and seq % 2 == 0 and d_word == 64
    mesh = plsc.VectorSubcoreMesh(core_axis_name="c", subcore_axis_name="s")

    @functools.partial(
        pl.kernel,
        mesh=mesh,
        compiler_params=pltpu.CompilerParams(needs_layout_passes=False),
        out_type=jax.ShapeDtypeStruct((seq, d_word, bsz), jnp.float32),
        scratch_types=[
            pltpu.VMEM((seq, 128), jnp.int32),
            pltpu.VMEM((4, 128), jnp.int32),
            pltpu.VMEM((4, 128, 128), jnp.float32),
            pltpu.VMEM((2, d_word, 128), jnp.float32),
            pltpu.SemaphoreType.DMA,
            pltpu.SemaphoreType.DMA,
            pltpu.SemaphoreType.DMA,
            pltpu.SemaphoreType.DMA,
            pltpu.SemaphoreType.DMA,
            pltpu.SemaphoreType.DMA,
        ],
    )
    def word_kernel(sent_t, w2, o_word, idx_v, pidx, rows, out_w,
                    sg0, sg1, sg2, sg3, so0, so1):
        wid = lax.axis_index("s") * 2 + lax.axis_index("c")
        bcol = wid * 128
        sem_g = (sg0, sg1, sg2, sg3)
        sem_o = (so0, so1)
        ii = lax.iota(jnp.int32, LANES)

        def prep_and_fire_gather(s, b):
            pidx_b = pidx.at[b]
            for jc in range(8):
                idx16 = idx_v[s, pl.ds(jc * LANES, LANES)]
                pidx_b[pl.ds(jc * LANES, LANES)] = lax.shift_right_logical(idx16, 1)
            pltpu.async_copy(w2.at[pidx.at[b]], rows.at[b], sem_g[b])

        def wait_gather(b):
            pltpu.make_async_copy(w2.at[pidx.at[b]], rows.at[b], sem_g[b]).wait()

        def transpose_word(s, b):
            rows_b = rows.at[b]
            out_b = out_w.at[b % 2]

            def jbody(jc, carry):
                j0 = jc * LANES
                idx16 = idx_v[s, pl.ds(j0, LANES)]
                col_base = (idx16 & 1) * d_word
                rvec = j0 + ii
                for d0 in range(0, d_word, 8):
                    gs = [plsc.load_gather(rows_b, [rvec, col_base + (d0 + k)])
                          for k in range(8)]
                    for k in range(8):
                        out_b[d0 + k, pl.ds(j0, LANES)] = gs[k]
                return carry

            lax.fori_loop(0, 8, jbody, 0)

        def wait_out(b, s):
            pltpu.make_async_copy(
                out_w.at[b], o_word.at[s, :, pl.ds(bcol, 128)], sem_o[b]).wait()

        pltpu.sync_copy(sent_t.at[:, pl.ds(bcol, 128)], idx_v)
        prep_and_fire_gather(0, 0)
        prep_and_fire_gather(1, 1)
        prep_and_fire_gather(2, 2)

        def outer(q, carry):
            for b in range(4):
                s = 4 * q + b

                @pl.when(s + 3 < seq)
                def _():
                    prep_and_fire_gather(s + 3, (b + 3) % 4)

                wait_gather(b)

                @pl.when(s >= 2)
                def _():
                    wait_out(b % 2, s)

                transpose_word(s, b)
                pltpu.async_copy(
                    out_w.at[b % 2], o_word.at[s, :, pl.ds(bcol, 128)],
                    sem_o[b % 2])
            return carry

        lax.fori_loop(0, seq // 4, outer, 0)
        for b in range(2):
            wait_out(b, 0)

    return word_kernel


def _make_small_kernel(bsz, seq, d_tag, d_rel):
    assert bsz == 128 * NW and seq % 2 == 0
    assert d_tag % 8 == 0 and d_rel % 8 == 0
    mesh = plsc.VectorSubcoreMesh(core_axis_name="c", subcore_axis_name="s")

    @functools.partial(
        pl.kernel,
        mesh=mesh,
        compiler_params=pltpu.CompilerParams(needs_layout_passes=False),
        out_type=(
            jax.ShapeDtypeStruct((seq, d_tag, bsz), jnp.float32),
            jax.ShapeDtypeStruct((seq, d_rel, bsz), jnp.float32),
        ),
        scratch_types=[
            pltpu.VMEM((seq, 128), jnp.int32),
            pltpu.VMEM((d_tag, 64), jnp.float32),
            pltpu.VMEM((2, d_tag, 128), jnp.float32),
            pltpu.SemaphoreType.DMA,
            pltpu.SemaphoreType.DMA,
        ],
    )
    def small_kernel(tag_t, rel_t, wtag_t, wrel_t, o_tag, o_rel,
                     idx_v, wtab, out_s, so0, so1):
        wid = lax.axis_index("s") * 2 + lax.axis_index("c")
        bcol = wid * 128
        sem_o = (so0, so1)

        def phase(idx_hbm, table_hbm, out_hbm, d_out):
            pltpu.sync_copy(idx_hbm.at[:, pl.ds(bcol, 128)], idx_v)
            pltpu.sync_copy(table_hbm, wtab)

            def wait_out(b, s):
                pltpu.make_async_copy(
                    out_s.at[b], out_hbm.at[s, :, pl.ds(bcol, 128)],
                    sem_o[b]).wait()

            def outer(o, carry):
                for b in range(2):
                    s = 2 * o + b

                    @pl.when(o >= 1)
                    def _():
                        wait_out(b, s)

                    out_b = out_s.at[b]

                    def jbody(jc, carry2):
                        j0 = jc * LANES
                        idx16 = idx_v[s, pl.ds(j0, LANES)]
                        for d0 in range(0, d_out, 8):
                            gs = [plsc.load_gather(
                                wtab,
                                [jnp.full((LANES,), d0 + k, jnp.int32), idx16])
                                for k in range(8)]
                            for k in range(8):
                                out_b[d0 + k, pl.ds(j0, LANES)] = gs[k]
                        return carry2

                    lax.fori_loop(0, 8, jbody, 0)
                    pltpu.async_copy(
                        out_s.at[b], out_hbm.at[s, :, pl.ds(bcol, 128)],
                        sem_o[b])
                return carry

            lax.fori_loop(0, seq // 2, outer, 0)
            for b in range(2):
                wait_out(b, 0)

        phase(tag_t, wtag_t, o_tag, d_tag)
        phase(rel_t, wrel_t, o_rel, d_rel)

    return small_kernel


def kernel(sent_inputs, tag_inputs, rel_inputs, W_word, W_tag, W_rel):
    bsz, seq = sent_inputs.shape
    n_vocab, d_word = W_word.shape
    d_tag = W_tag.shape[1]
    d_rel = W_rel.shape[1]
    pack = 128 // d_word
    w2 = W_word.reshape(n_vocab // pack, 128)
    small_fn = _make_small_kernel(bsz, seq, d_tag, d_rel)
    word_fn = _make_word_kernel(bsz, seq, d_word)
    o_t, o_r = small_fn(
        tag_inputs.T.astype(jnp.int32),
        rel_inputs.T.astype(jnp.int32),
        W_tag.T, W_rel.T)
    o_w = word_fn(sent_inputs.T.astype(jnp.int32), w2)
    return (
        jnp.expand_dims(jnp.transpose(o_w, (2, 0, 1)), 1),
        jnp.expand_dims(jnp.transpose(o_t, (2, 0, 1)), 1),
        jnp.expand_dims(jnp.transpose(o_r, (2, 0, 1)), 1),
    )


# speedup vs baseline: 1.0095x; 1.0095x over previous
import functools

import jax
import jax.numpy as jnp
from jax import lax
from jax.experimental import pallas as pl
from jax.experimental.pallas import tpu as pltpu
from jax.experimental.pallas import tpu_sc as plsc

NW = 32
LANES = 16


def _make_word_kernel(bsz, seq, d_word):
    assert bsz == 128 * NW and seq % 4 == 0 and d_word == 64
    mesh = plsc.VectorSubcoreMesh(core_axis_name="c", subcore_axis_name="s")

    @functools.partial(
        pl.kernel,
        mesh=mesh,
        compiler_params=pltpu.CompilerParams(
            use_tc_tiling_on_sc=False, needs_layout_passes=False),
        out_type=jax.ShapeDtypeStruct((seq, d_word // 8, NW, 8, 128),
                                      jnp.float32),
        scratch_types=[
            pltpu.VMEM((seq, 128), jnp.int32),
            pltpu.VMEM((4, 128, d_word), jnp.float32),
            pltpu.VMEM((2, d_word // 8, 8, 128), jnp.float32),
            pltpu.SemaphoreType.DMA,
            pltpu.SemaphoreType.DMA,
            pltpu.SemaphoreType.DMA,
            pltpu.SemaphoreType.DMA,
            pltpu.SemaphoreType.DMA,
            pltpu.SemaphoreType.DMA,
        ],
    )
    def word_kernel(sent_t, wt, o5, idx_v, rows, out_w,
                    sg0, sg1, sg2, sg3, so0, so1):
        wid = lax.axis_index("s") * 2 + lax.axis_index("c")
        bcol = wid * 128
        sem_g = (sg0, sg1, sg2, sg3)
        sem_o = (so0, so1)
        ii = lax.iota(jnp.int32, LANES)

        def fire_gather(s, b):
            pltpu.async_copy(wt.at[idx_v.at[s]], rows.at[b], sem_g[b])

        def wait_gather(s, b):
            pltpu.make_async_copy(wt.at[idx_v.at[s]], rows.at[b],
                                  sem_g[b]).wait()

        def transpose_word(s, b):
            rows_b = rows.at[b]
            out_b = out_w.at[b % 2]

            def jbody(jc, carry):
                j0 = jc * LANES
                rvec = j0 + ii
                for d0 in range(0, d_word, 8):
                    gs = [plsc.load_gather(
                        rows_b, [rvec, jnp.full((LANES,), d0 + k, jnp.int32)])
                        for k in range(8)]
                    for k in range(8):
                        d = d0 + k
                        out_b[d // 8, d % 8, pl.ds(j0, LANES)] = gs[k]
                return carry

            lax.fori_loop(0, 8, jbody, 0)

        def wait_out(b, s):
            pltpu.make_async_copy(
                out_w.at[b], o5.at[s, :, wid], sem_o[b]).wait()

        pltpu.sync_copy(sent_t.at[:, pl.ds(bcol, 128)], idx_v)
        fire_gather(0, 0)
        fire_gather(1, 1)
        fire_gather(2, 2)

        def outer(q, carry):
            for b in range(4):
                s = 4 * q + b

                @pl.when(s + 3 < seq)
                def _():
                    fire_gather(s + 3, (b + 3) % 4)

                wait_gather(s, b)

                @pl.when(s >= 2)
                def _():
                    wait_out(b % 2, s)

                transpose_word(s, b)
                pltpu.async_copy(out_w.at[b % 2], o5.at[s, :, wid],
                                 sem_o[b % 2])
            return carry

        lax.fori_loop(0, seq // 4, outer, 0)
        for b in range(2):
            wait_out(b, 0)

    return word_kernel


def _make_small_kernel(bsz, seq, d_tag, d_rel):
    assert bsz == 128 * NW and seq % 2 == 0
    assert d_tag % 8 == 0 and d_rel % 8 == 0
    mesh = plsc.VectorSubcoreMesh(core_axis_name="c", subcore_axis_name="s")

    @functools.partial(
        pl.kernel,
        mesh=mesh,
        compiler_params=pltpu.CompilerParams(needs_layout_passes=False),
        out_type=(
            jax.ShapeDtypeStruct((seq, d_tag, bsz), jnp.float32),
            jax.ShapeDtypeStruct((seq, d_rel, bsz), jnp.float32),
        ),
        scratch_types=[
            pltpu.VMEM((seq, 128), jnp.int32),
            pltpu.VMEM((d_tag, 64), jnp.float32),
            pltpu.VMEM((2, d_tag, 128), jnp.float32),
            pltpu.SemaphoreType.DMA,
            pltpu.SemaphoreType.DMA,
        ],
    )
    def small_kernel(tag_t, rel_t, wtag_t, wrel_t, o_tag, o_rel,
                     idx_v, wtab, out_s, so0, so1):
        wid = lax.axis_index("s") * 2 + lax.axis_index("c")
        bcol = wid * 128
        sem_o = (so0, so1)

        def phase(idx_hbm, table_hbm, out_hbm, d_out):
            pltpu.sync_copy(idx_hbm.at[:, pl.ds(bcol, 128)], idx_v)
            pltpu.sync_copy(table_hbm, wtab)

            def wait_out(b, s):
                pltpu.make_async_copy(
                    out_s.at[b], out_hbm.at[s, :, pl.ds(bcol, 128)],
                    sem_o[b]).wait()

            def outer(o, carry):
                for b in range(2):
                    s = 2 * o + b

                    @pl.when(o >= 1)
                    def _():
                        wait_out(b, s)

                    out_b = out_s.at[b]

                    def jbody(jc, carry2):
                        j0 = jc * LANES
                        idx16 = idx_v[s, pl.ds(j0, LANES)]
                        for d0 in range(0, d_out, 8):
                            gs = [plsc.load_gather(
                                wtab,
                                [jnp.full((LANES,), d0 + k, jnp.int32), idx16])
                                for k in range(8)]
                            for k in range(8):
                                out_b[d0 + k, pl.ds(j0, LANES)] = gs[k]
                        return carry2

                    lax.fori_loop(0, 8, jbody, 0)
                    pltpu.async_copy(
                        out_s.at[b], out_hbm.at[s, :, pl.ds(bcol, 128)],
                        sem_o[b])
                return carry

            lax.fori_loop(0, seq // 2, outer, 0)
            for b in range(2):
                wait_out(b, 0)

        phase(tag_t, wtag_t, o_tag, d_tag)
        phase(rel_t, wrel_t, o_rel, d_rel)

    return small_kernel


def kernel(sent_inputs, tag_inputs, rel_inputs, W_word, W_tag, W_rel):
    bsz, seq = sent_inputs.shape
    n_vocab, d_word = W_word.shape
    d_tag = W_tag.shape[1]
    d_rel = W_rel.shape[1]
    small_fn = _make_small_kernel(bsz, seq, d_tag, d_rel)
    word_fn = _make_word_kernel(bsz, seq, d_word)
    o_t, o_r = small_fn(
        tag_inputs.T.astype(jnp.int32),
        rel_inputs.T.astype(jnp.int32),
        W_tag.T, W_rel.T)
    o5 = word_fn(sent_inputs.T.astype(jnp.int32), W_word)
    o_w = jnp.transpose(o5, (2, 4, 0, 1, 3)).reshape(bsz, seq, d_word)
    return (
        jnp.expand_dims(o_w, 1),
        jnp.expand_dims(jnp.transpose(o_t, (2, 0, 1)), 1),
        jnp.expand_dims(jnp.transpose(o_r, (2, 0, 1)), 1),
    )
